# baseline (speedup 1.0000x reference)
import functools
import math

import jax
import jax.numpy as jnp
from jax import lax
from jax.experimental import pallas as pl
from jax.experimental.pallas import tpu as pltpu
from jax.experimental.pallas import tpu_sc as plsc


def _sc_row_gather(table, idx):
    s_len, row = table.shape
    info = plsc.get_sparse_core_info()
    nw = info.num_cores * info.num_subcores
    b_per_w = s_len // nw

    mesh = plsc.VectorSubcoreMesh(core_axis_name="c", subcore_axis_name="s")

    @functools.partial(
        pl.kernel,
        mesh=mesh,
        out_type=jax.ShapeDtypeStruct((s_len, row), jnp.float32),
        scratch_types=[
            pltpu.VMEM((b_per_w,), jnp.int32),
            pltpu.VMEM((b_per_w, row), jnp.float32),
            pltpu.SemaphoreType.DMA,
        ],
    )
    def gather_kernel(table_hbm, idx_hbm, out_hbm, idx_v, rows_v, sem):
        wid = lax.axis_index("s") * info.num_cores + lax.axis_index("c")
        base = wid * b_per_w
        pltpu.sync_copy(idx_hbm.at[pl.ds(base, b_per_w)], idx_v)
        pltpu.async_copy(table_hbm.at[idx_v], rows_v, sem).wait()
        pltpu.sync_copy(rows_v, out_hbm.at[pl.ds(base, b_per_w)])

    return gather_kernel(table, idx)


def _ctx_body(q_ref, k_ref, v_ref, gm_ref, rm_ref, ctx_ref, *, inv_scale):
    qb16 = q_ref[0].astype(jnp.bfloat16)
    kk16 = k_ref[0].astype(jnp.bfloat16)
    gm = gm_ref[...]
    rm = rm_ref[...]

    s_full = lax.dot_general(qb16, kk16, (((1,), (1,)), ((), ())),
                             preferred_element_type=jnp.float32) * inv_scale
    m = jnp.max(jnp.where(gm > 0.0, s_full, -jnp.inf), axis=1, keepdims=True)
    e = jnp.exp(s_full - m)
    den_all = jnp.sum(e, axis=1, keepdims=True)
    eg = e * gm
    den_g = jnp.sum(eg, axis=1, keepdims=True)
    p = jnp.where(rm > 0.0, e * (1.0 / den_all), eg * (1.0 / den_g))
    ctx_ref[0] = lax.dot_general(p.astype(jnp.bfloat16),
                                 v_ref[0].astype(jnp.bfloat16),
                                 (((1,), (0,)), ((), ())),
                                 preferred_element_type=jnp.float32)


def _probs_body(q_ref, gk_ref, vc_ref, probs_ref, *, inv_scale):
    qb16 = q_ref[0].astype(jnp.bfloat16)
    gk16 = gk_ref[0].astype(jnp.bfloat16)
    vc = vc_ref[...]

    s = lax.dot_general(qb16, gk16, (((1,), (1,)), ((), ())),
                        preferred_element_type=jnp.float32) * inv_scale
    s = jnp.where(vc > 0.0, s, -jnp.inf)
    m = jnp.max(s, axis=1, keepdims=True)
    e = jnp.exp(s - m)
    probs_ref[0] = e * (1.0 / jnp.sum(e, axis=1, keepdims=True))


def kernel(q, k, v, numeric_embedding_manager, attention_mask):
    B, H, S, D = q.shape
    BH = B * H
    q2 = q.reshape(BH, S, D)
    k2 = k.reshape(BH, S, D)
    v2 = v.reshape(BH, S, D)

    is_g = (attention_mask > 0)
    order = jnp.argsort(jnp.logical_not(is_g).astype(jnp.int32),
                        axis=1, stable=True).astype(jnp.int32)
    n_glob = is_g.sum(axis=1).astype(jnp.int32)

    kt = k2.transpose(1, 0, 2).reshape(S, BH * D)
    gk_t = _sc_row_gather(kt, order.reshape(S))
    gk = gk_t.reshape(S, BH, D).transpose(1, 0, 2)

    pos = jnp.arange(S, dtype=jnp.int32)
    valid_col = (pos[None, :] < n_glob[:, None]).astype(jnp.float32)
    gmask_col = is_g.astype(jnp.float32)
    rmask = gmask_col[0][:, None]

    bq = 256
    grid = (BH, S // bq)
    row_block = pl.BlockSpec((1, bq, D), lambda h, i: (h, i, 0))
    full_block = pl.BlockSpec((1, S, D), lambda h, i: (h, 0, 0))
    col_block = pl.BlockSpec((1, S), lambda h, i: (0, 0))
    probs_spec = pl.BlockSpec((1, bq, S), lambda h, i: (h, i, 0))
    rm_spec = pl.BlockSpec((bq, 1), lambda h, i: (i, 0))
    params = pltpu.CompilerParams(
        dimension_semantics=("arbitrary", "arbitrary"))

    ctx = pl.pallas_call(
        functools.partial(_ctx_body, inv_scale=1.0 / math.sqrt(D)),
        grid=grid,
        in_specs=[row_block, full_block, full_block, col_block, rm_spec],
        out_specs=row_block,
        out_shape=jax.ShapeDtypeStruct((BH, S, D), jnp.float32),
        compiler_params=params,
    )(q2, k2, v2, gmask_col, rmask)

    probs = pl.pallas_call(
        functools.partial(_probs_body, inv_scale=1.0 / math.sqrt(D)),
        grid=grid,
        in_specs=[row_block, full_block, col_block],
        out_specs=probs_spec,
        out_shape=jax.ShapeDtypeStruct((BH, S, S), jnp.float32),
        compiler_params=params,
    )(q2, gk, valid_col)

    return ctx.reshape(B, H, S, D), probs.reshape(B, H, S, S)


# speedup vs baseline: 1.2044x; 1.2044x over previous
import functools
import math

import jax
import jax.numpy as jnp
from jax import lax
from jax.experimental import pallas as pl
from jax.experimental.pallas import tpu as pltpu
from jax.experimental.pallas import tpu_sc as plsc


def _sc_row_gather(table, idx):
    s_len, row = table.shape
    info = plsc.get_sparse_core_info()
    nw = info.num_cores * info.num_subcores
    b_per_w = s_len // nw

    mesh = plsc.VectorSubcoreMesh(core_axis_name="c", subcore_axis_name="s")

    @functools.partial(
        pl.kernel,
        mesh=mesh,
        out_type=jax.ShapeDtypeStruct((s_len, row), jnp.float32),
        scratch_types=[
            pltpu.VMEM((b_per_w,), jnp.int32),
            pltpu.VMEM((b_per_w, row), jnp.float32),
            pltpu.SemaphoreType.DMA,
        ],
    )
    def gather_kernel(table_hbm, idx_hbm, out_hbm, idx_v, rows_v, sem):
        wid = lax.axis_index("s") * info.num_cores + lax.axis_index("c")
        base = wid * b_per_w
        pltpu.sync_copy(idx_hbm.at[pl.ds(base, b_per_w)], idx_v)
        pltpu.async_copy(table_hbm.at[idx_v], rows_v, sem).wait()
        pltpu.sync_copy(rows_v, out_hbm.at[pl.ds(base, b_per_w)])

    return gather_kernel(table, idx)


def _attn_body(q_ref, k_ref, gk_ref, v_ref, gm_ref, vc_ref, rm_ref,
               probs_ref, ctx_ref):
    qb16 = q_ref[0].astype(jnp.bfloat16)
    gm = gm_ref[...]
    vc = vc_ref[...]
    rm = rm_ref[...]

    dims = (((1,), (1,)), ((), ()))
    s_glob = lax.dot_general(qb16, gk_ref[0].astype(jnp.bfloat16), dims,
                             preferred_element_type=jnp.float32)
    s_glob = jnp.where(vc > 0.0, s_glob, -jnp.inf)
    m_gl = jnp.max(s_glob, axis=1, keepdims=True)
    e_gl = jnp.exp(s_glob - m_gl)

    s_full = lax.dot_general(qb16, k_ref[0].astype(jnp.bfloat16), dims,
                             preferred_element_type=jnp.float32)
    e = jnp.exp(s_full - m_gl)
    eg = e * gm
    den_all = jnp.sum(e, axis=1, keepdims=True)
    den_g = jnp.sum(eg, axis=1, keepdims=True)

    probs_ref[0] = e_gl * (1.0 / den_g)

    den = jnp.where(rm > 0.0, den_all, den_g)
    p_ctx = jnp.where(rm > 0.0, e, eg) * (1.0 / den)
    ctx_ref[0] = lax.dot_general(p_ctx.astype(jnp.bfloat16),
                                 v_ref[0].astype(jnp.bfloat16),
                                 (((1,), (0,)), ((), ())),
                                 preferred_element_type=jnp.float32)


def kernel(q, k, v, numeric_embedding_manager, attention_mask):
    B, H, S, D = q.shape
    BH = B * H
    q2 = (q * (1.0 / math.sqrt(D))).reshape(BH, S, D)
    k2 = k.reshape(BH, S, D)
    v2 = v.reshape(BH, S, D)

    is_g = (attention_mask > 0)
    order = jnp.argsort(jnp.logical_not(is_g).astype(jnp.int32),
                        axis=1, stable=True).astype(jnp.int32)
    n_glob = is_g.sum(axis=1).astype(jnp.int32)

    kt = k2.transpose(1, 0, 2).reshape(S, BH * D)
    gk_t = _sc_row_gather(kt, order.reshape(S))
    gk = gk_t.reshape(S, BH, D).transpose(1, 0, 2)

    pos = jnp.arange(S, dtype=jnp.int32)
    valid_col = (pos[None, :] < n_glob[:, None]).astype(jnp.float32)
    gmask_col = is_g.astype(jnp.float32)
    rmask = gmask_col[0][:, None]

    bq = 256
    grid = (BH, S // bq)
    row_block = pl.BlockSpec((1, bq, D), lambda h, i: (h, i, 0))
    full_block = pl.BlockSpec((1, S, D), lambda h, i: (h, 0, 0))
    col_block = pl.BlockSpec((1, S), lambda h, i: (0, 0))
    probs_spec = pl.BlockSpec((1, bq, S), lambda h, i: (h, i, 0))
    rm_spec = pl.BlockSpec((bq, 1), lambda h, i: (i, 0))

    probs, ctx = pl.pallas_call(
        _attn_body,
        grid=grid,
        in_specs=[row_block, full_block, full_block, full_block,
                  col_block, col_block, rm_spec],
        out_specs=[probs_spec, row_block],
        out_shape=[
            jax.ShapeDtypeStruct((BH, S, S), jnp.float32),
            jax.ShapeDtypeStruct((BH, S, D), jnp.float32),
        ],
        compiler_params=pltpu.CompilerParams(
            dimension_semantics=("arbitrary", "arbitrary"),
        ),
    )(q2, k2, gk, v2, gmask_col, valid_col, rmask)

    return ctx.reshape(B, H, S, D), probs.reshape(B, H, S, S)


# speedup vs baseline: 1.2996x; 1.0791x over previous
import functools
import math

import jax
import jax.numpy as jnp
from jax import lax
from jax.experimental import pallas as pl
from jax.experimental.pallas import tpu as pltpu
from jax.experimental.pallas import tpu_sc as plsc


def _sc_compact_rows(kvt, dest):
    s_len, row = kvt.shape
    info = plsc.get_sparse_core_info()
    nw = info.num_cores * info.num_subcores
    b_per_w = s_len // nw

    mesh = plsc.VectorSubcoreMesh(core_axis_name="c", subcore_axis_name="s")

    @functools.partial(
        pl.kernel,
        mesh=mesh,
        out_type=jax.ShapeDtypeStruct((s_len, row), jnp.float32),
        scratch_types=[
            pltpu.VMEM((b_per_w,), jnp.int32),
            pltpu.VMEM((b_per_w, row), jnp.float32),
            pltpu.SemaphoreType.DMA,
            pltpu.SemaphoreType.DMA,
        ],
    )
    def scatter_kernel(kvt_hbm, dest_hbm, out_hbm, idx_v, rows, rsem, osem):
        wid = lax.axis_index("s") * info.num_cores + lax.axis_index("c")
        base = wid * b_per_w
        cr = pltpu.async_copy(kvt_hbm.at[pl.ds(base, b_per_w)], rows, rsem)
        pltpu.sync_copy(dest_hbm.at[pl.ds(base, b_per_w)], idx_v)
        cr.wait()
        pltpu.async_copy(rows, out_hbm.at[idx_v], osem).wait()

    return scatter_kernel(kvt, dest)


def _attn_body(q_ref, gk_ref, gv_ref, vca_ref, vc_ref, rm_ref,
               probs_ref, ctx_ref):
    qb = q_ref[0]
    vca = vca_ref[...]
    vc = vc_ref[...]
    rm = rm_ref[...]

    s = lax.dot_general(qb, gk_ref[0].astype(jnp.bfloat16), (((1,), (1,)), ((), ())),
                        preferred_element_type=jnp.float32)
    m = jnp.max(s + vca, axis=1, keepdims=True)
    e_all = jnp.exp(s - m)
    e_m = e_all * vc
    den_all = jnp.sum(e_all, axis=1, keepdims=True)
    den_g = jnp.sum(e_m, axis=1, keepdims=True)

    probs_ref[0] = e_m * (1.0 / den_g)

    den = jnp.where(rm > 0.0, den_all, den_g)
    p = jnp.where(rm > 0.0, e_all, e_m) * (1.0 / den)
    ctx_ref[0] = lax.dot_general(p.astype(jnp.bfloat16),
                                 gv_ref[0].astype(jnp.bfloat16),
                                 (((1,), (0,)), ((), ())),
                                 preferred_element_type=jnp.float32)


def kernel(q, k, v, numeric_embedding_manager, attention_mask):
    B, H, S, D = q.shape
    BH = B * H
    qs = (q.reshape(BH, S, D) * (1.0 / math.sqrt(D))).astype(jnp.bfloat16)

    isg = attention_mask[0] > 0
    n_glob = isg.sum().astype(jnp.int32)
    c = jnp.cumsum(isg.astype(jnp.int32))
    pos = jnp.arange(S, dtype=jnp.int32)
    dest = jnp.where(isg, c - 1, n_glob + pos - c).astype(jnp.int32)

    kv = jnp.concatenate([k.reshape(BH, S, D), v.reshape(BH, S, D)], axis=0)
    kvt = kv.transpose(1, 0, 2).reshape(S, 2 * BH * D)
    gkv_t = _sc_compact_rows(kvt, dest)
    gkv = gkv_t.reshape(S, 2 * BH, D).transpose(1, 0, 2)
    gk = gkv[:BH]
    gv = gkv[BH:]

    valid = (pos[None, :] < n_glob).astype(jnp.float32)
    vca = (1.0 - valid) * jnp.float32(-1e30)
    rmask = isg.astype(jnp.float32)[:, None]

    bq = 256
    grid = (BH, S // bq)
    row_block = pl.BlockSpec((1, bq, D), lambda h, i: (h, i, 0))
    full_block = pl.BlockSpec((1, S, D), lambda h, i: (h, 0, 0))
    col_block = pl.BlockSpec((1, S), lambda h, i: (0, 0))
    probs_spec = pl.BlockSpec((1, bq, S), lambda h, i: (h, i, 0))
    rm_spec = pl.BlockSpec((bq, 1), lambda h, i: (i, 0))

    probs, ctx = pl.pallas_call(
        _attn_body,
        grid=grid,
        in_specs=[row_block, full_block, full_block,
                  col_block, col_block, rm_spec],
        out_specs=[probs_spec, row_block],
        out_shape=[
            jax.ShapeDtypeStruct((BH, S, S), jnp.float32),
            jax.ShapeDtypeStruct((BH, S, D), jnp.float32),
        ],
        compiler_params=pltpu.CompilerParams(
            dimension_semantics=("arbitrary", "arbitrary"),
        ),
    )(qs, gk, gv, vca, valid, rmask)

    return ctx.reshape(B, H, S, D), probs.reshape(B, H, S, S)
